# Initial kernel scaffold; baseline (speedup 1.0000x reference)
import jax
import jax.numpy as jnp
from jax.experimental import pallas as pl


def kernel(x, table):
    raise NotImplementedError("write your pallas kernel here")


# speedup vs baseline: 1.4386x; 1.4386x over previous
import functools

import jax
import jax.numpy as jnp
from jax import lax
from jax.experimental import pallas as pl
from jax.experimental.pallas import tpu as pltpu
from jax.experimental.pallas import tpu_sc as plsc

_CH = 128


@functools.lru_cache(maxsize=None)
def _make_kernel(B, D, NW, NC):
    b_per_w = B // NW
    n_chunks = b_per_w // _CH
    mesh = plsc.VectorSubcoreMesh(core_axis_name="c", subcore_axis_name="s")

    @functools.partial(
        pl.kernel,
        mesh=mesh,
        compiler_params=pltpu.CompilerParams(use_tc_tiling_on_sc=False),
        out_type=jax.ShapeDtypeStruct((B, D), jnp.float32),
        scratch_types=[
            pltpu.VMEM((n_chunks, _CH), jnp.int32),
            pltpu.VMEM((_CH, D), jnp.float32),
            pltpu.SemaphoreType.DMA,
        ],
    )
    def k(idx_hbm, table_hbm, out_hbm, idx_v, rows_v, gsem):
        wid = lax.axis_index("s") * NC + lax.axis_index("c")
        base = wid * b_per_w
        pltpu.sync_copy(idx_hbm.at[wid], idx_v)

        def body(j, carry):
            pltpu.async_copy(table_hbm.at[idx_v.at[j]], rows_v, gsem).wait()
            pltpu.sync_copy(rows_v, out_hbm.at[pl.ds(base + j * _CH, _CH)])
            return carry

        lax.fori_loop(0, n_chunks, body, 0)

    return k


def kernel(x, table):
    B0, S = x.shape
    V, D = table.shape
    B = B0 * S
    info = plsc.get_sparse_core_info()
    NW = info.num_cores * info.num_subcores
    idx = x.reshape(NW, (B // NW) // _CH, _CH).astype(jnp.int32)
    out = _make_kernel(B, D, NW, info.num_cores)(idx, table)
    return out.reshape(B0, S, D)


# speedup vs baseline: 1.5735x; 1.0938x over previous
import functools

import jax
import jax.numpy as jnp
from jax import lax
from jax.experimental import pallas as pl
from jax.experimental.pallas import tpu as pltpu
from jax.experimental.pallas import tpu_sc as plsc

_CH = 128
_K = 13


@functools.lru_cache(maxsize=None)
def _make_kernel(B, D, NW, NC):
    b_per_w = B // NW
    n_chunks = b_per_w // _CH
    n_phases = n_chunks // _K
    rows_per_phase = _K * _CH
    mesh = plsc.VectorSubcoreMesh(core_axis_name="c", subcore_axis_name="s")

    @functools.partial(
        pl.kernel,
        mesh=mesh,
        compiler_params=pltpu.CompilerParams(use_tc_tiling_on_sc=False),
        out_type=jax.ShapeDtypeStruct((B, D), jnp.float32),
        scratch_types=[
            pltpu.VMEM((n_chunks, _CH), jnp.int32),
            pltpu.VMEM((2, rows_per_phase, D), jnp.float32),
            pltpu.SemaphoreType.DMA,
            pltpu.SemaphoreType.DMA,
            pltpu.SemaphoreType.DMA,
            pltpu.SemaphoreType.DMA,
        ],
    )
    def k(idx_hbm, table_hbm, out_hbm, idx_v, rows_v, g0, g1, o0, o1):
        gsem = (g0, g1)
        osem = (o0, o1)
        wid = lax.axis_index("s") * NC + lax.axis_index("c")
        base = wid * b_per_w
        pltpu.sync_copy(idx_hbm.at[wid], idx_v)

        def fire(phase, buf):
            for c in range(_K):
                pltpu.async_copy(
                    table_hbm.at[idx_v.at[phase * _K + c]],
                    rows_v.at[buf].at[pl.ds(c * _CH, _CH)],
                    gsem[buf],
                )

        def drain_gathers(buf):
            for c in range(_K):
                pltpu.make_async_copy(
                    table_hbm.at[idx_v.at[0]],
                    rows_v.at[buf].at[pl.ds(c * _CH, _CH)],
                    gsem[buf],
                ).wait()

        fire(0, 0)

        def group(g, carry):
            for b in (0, 1):
                p = 2 * g + b
                nb = 1 - b
                if b == 0:
                    @pl.when(g > 0)
                    def _():
                        pltpu.make_async_copy(
                            rows_v.at[nb],
                            out_hbm.at[pl.ds(base, rows_per_phase)],
                            osem[nb],
                        ).wait()

                    fire(p + 1, nb)
                else:
                    pltpu.make_async_copy(
                        rows_v.at[nb],
                        out_hbm.at[pl.ds(base, rows_per_phase)],
                        osem[nb],
                    ).wait()

                    @pl.when(g < n_phases // 2 - 1)
                    def _():
                        fire(p + 1, nb)

                drain_gathers(b)
                pltpu.async_copy(
                    rows_v.at[b],
                    out_hbm.at[pl.ds(base + p * rows_per_phase, rows_per_phase)],
                    osem[b],
                )
            return carry

        lax.fori_loop(0, n_phases // 2, group, 0)
        pltpu.make_async_copy(
            rows_v.at[1],
            out_hbm.at[pl.ds(base, rows_per_phase)],
            osem[1],
        ).wait()

    return k


def kernel(x, table):
    B0, S = x.shape
    V, D = table.shape
    B = B0 * S
    info = plsc.get_sparse_core_info()
    NW = info.num_cores * info.num_subcores
    idx = x.reshape(NW, (B // NW) // _CH, _CH).astype(jnp.int32)
    out = _make_kernel(B, D, NW, info.num_cores)(idx, table)
    return out.reshape(B0, S, D)
